# Initial kernel scaffold; baseline (speedup 1.0000x reference)
import jax
import jax.numpy as jnp
from jax.experimental import pallas as pl


def kernel(word, glove_table, fasttext_table, Wg, bg, Wf, bf, Wa1, ba1, Wa2, ba2):
    raise NotImplementedError("write your pallas kernel here")


# speedup vs baseline: 1.7068x; 1.7068x over previous
import jax
import jax.numpy as jnp
from jax import lax
from jax.experimental import pallas as pl
from jax.experimental.pallas import tpu as pltpu
from jax.experimental.pallas import tpu_sc as plsc

V = 100000
D = 300
DP = 384
N_TOK = 51200

NW = 32
PER_W = N_TOK // NW
CHUNK = 80
NCHUNK = PER_W // CHUNK

BT = 512


def _gather_body(gt_hbm, ft_hbm, idx_hbm, outg_hbm, outf_hbm,
                 idx_v, bufg, buff, semg, semf):
    wid = lax.axis_index("s") * 2 + lax.axis_index("c")
    base = wid * PER_W
    pltpu.sync_copy(idx_hbm.at[pl.ds(base, PER_W)], idx_v)

    def step(c, carry):
        off = pl.multiple_of(c * CHUNK, 8)
        idx_c = idx_v.at[pl.ds(off, CHUNK)]
        cg = pltpu.async_copy(gt_hbm.at[idx_c], bufg, semg)
        cf = pltpu.async_copy(ft_hbm.at[idx_c], buff, semf)
        cg.wait()
        pltpu.sync_copy(bufg, outg_hbm.at[pl.ds(base + off, CHUNK)])
        cf.wait()
        pltpu.sync_copy(buff, outf_hbm.at[pl.ds(base + off, CHUNK)])
        return carry

    lax.fori_loop(0, NCHUNK, step, 0)


def _sc_gather(glove_pad, fasttext_pad, idx):
    mesh = plsc.VectorSubcoreMesh(core_axis_name="c", subcore_axis_name="s")
    f = pl.kernel(
        _gather_body,
        out_type=(
            jax.ShapeDtypeStruct((N_TOK, DP), jnp.float32),
            jax.ShapeDtypeStruct((N_TOK, DP), jnp.float32),
        ),
        mesh=mesh,
        scratch_types=[
            pltpu.VMEM((PER_W,), jnp.int32),
            pltpu.VMEM((CHUNK, DP), jnp.float32),
            pltpu.VMEM((CHUNK, DP), jnp.float32),
            pltpu.SemaphoreType.DMA,
            pltpu.SemaphoreType.DMA,
        ],
    )
    return f(glove_pad, fasttext_pad, idx)


def _combine_body(g_ref, f_ref, wg_ref, bg_ref, wf_ref, bf_ref,
                  wa1_ref, wa2_ref, out_ref):
    g = g_ref[:, :D]
    f = f_ref[:, :D]
    dn = (((1,), (1,)), ((), ()))
    g_out = lax.dot_general(g, wg_ref[...], dn,
                            preferred_element_type=jnp.float32) + bg_ref[...]
    f_out = lax.dot_general(f, wf_ref[...], dn,
                            preferred_element_type=jnp.float32) + bf_ref[...]
    wvec = lax.dot_general(wa2_ref[...], wa1_ref[...], (((1,), (0,)), ((), ())),
                           preferred_element_type=jnp.float32)
    diff = jnp.sum((g_out - f_out) * wvec, axis=1, keepdims=True)
    s = 1.0 / (1.0 + jnp.exp(-diff))
    out_ref[...] = jnp.maximum(s * g_out + (1.0 - s) * f_out, 0.0)


def _tc_combine(g_emb, f_emb, Wg, bg, Wf, bf, Wa1, Wa2):
    grid = (N_TOK // BT,)
    return pl.pallas_call(
        _combine_body,
        grid=grid,
        in_specs=[
            pl.BlockSpec((BT, DP), lambda i: (i, 0)),
            pl.BlockSpec((BT, DP), lambda i: (i, 0)),
            pl.BlockSpec((D, D), lambda i: (0, 0)),
            pl.BlockSpec((1, D), lambda i: (0, 0)),
            pl.BlockSpec((D, D), lambda i: (0, 0)),
            pl.BlockSpec((1, D), lambda i: (0, 0)),
            pl.BlockSpec((10, D), lambda i: (0, 0)),
            pl.BlockSpec((1, 10), lambda i: (0, 0)),
        ],
        out_specs=pl.BlockSpec((BT, D), lambda i: (i, 0)),
        out_shape=jax.ShapeDtypeStruct((N_TOK, D), jnp.float32),
    )(g_emb, f_emb, Wg, bg, Wf, bf, Wa1, Wa2)


def kernel(word, glove_table, fasttext_table, Wg, bg, Wf, bf, Wa1, ba1, Wa2, ba2):
    B, L = word.shape
    idx = word.reshape(-1).astype(jnp.int32)
    gtp = jnp.pad(glove_table, ((0, 0), (0, DP - D)))
    ftp = jnp.pad(fasttext_table, ((0, 0), (0, DP - D)))
    g_emb, f_emb = _sc_gather(gtp, ftp, idx)
    out = _tc_combine(g_emb, f_emb, Wg, bg.reshape(1, D), Wf, bf.reshape(1, D),
                      Wa1, Wa2)
    return out.reshape(B, L, D)


# speedup vs baseline: 3.2620x; 1.9112x over previous
import jax
import jax.numpy as jnp
from jax import lax
from jax.experimental import pallas as pl
from jax.experimental.pallas import tpu as pltpu
from jax.experimental.pallas import tpu_sc as plsc

V = 100000
D = 300
DH = 256
DT = 128
D_TAIL = D - DH
N_TOK = 51200

NW = 32
PER_W = N_TOK // NW
CHUNK = 80
NCHUNK = PER_W // CHUNK

BT = 512


def _gather_body(gt_hbm, ft_hbm, gtail_hbm, ftail_hbm, idx_hbm,
                 outg_hbm, outgt_hbm, outf_hbm, outft_hbm,
                 idx_v, bufg, bufgt, buff, bufft, semg, semf):
    wid = lax.axis_index("s") * 2 + lax.axis_index("c")
    base = wid * PER_W
    pltpu.sync_copy(idx_hbm.at[pl.ds(base, PER_W)], idx_v)

    def step(c, carry):
        off = pl.multiple_of(c * CHUNK, 8)
        idx_c = idx_v.at[pl.ds(off, CHUNK)]
        cg = pltpu.async_copy(gt_hbm.at[:, pl.ds(0, DH)].at[idx_c], bufg, semg)
        cgt = pltpu.async_copy(gtail_hbm.at[idx_c], bufgt, semg)
        cf = pltpu.async_copy(ft_hbm.at[:, pl.ds(0, DH)].at[idx_c], buff, semf)
        cft = pltpu.async_copy(ftail_hbm.at[idx_c], bufft, semf)
        cg.wait()
        pltpu.sync_copy(bufg, outg_hbm.at[pl.ds(base + off, CHUNK)])
        cgt.wait()
        pltpu.sync_copy(bufgt, outgt_hbm.at[pl.ds(base + off, CHUNK)])
        cf.wait()
        pltpu.sync_copy(buff, outf_hbm.at[pl.ds(base + off, CHUNK)])
        cft.wait()
        pltpu.sync_copy(bufft, outft_hbm.at[pl.ds(base + off, CHUNK)])
        return carry

    lax.fori_loop(0, NCHUNK, step, 0)


def _sc_gather(gt, ft, gtail, ftail, idx):
    mesh = plsc.VectorSubcoreMesh(core_axis_name="c", subcore_axis_name="s")
    f = pl.kernel(
        _gather_body,
        out_type=(
            jax.ShapeDtypeStruct((N_TOK, DH), jnp.float32),
            jax.ShapeDtypeStruct((N_TOK, DT), jnp.float32),
            jax.ShapeDtypeStruct((N_TOK, DH), jnp.float32),
            jax.ShapeDtypeStruct((N_TOK, DT), jnp.float32),
        ),
        mesh=mesh,
        scratch_types=[
            pltpu.VMEM((PER_W,), jnp.int32),
            pltpu.VMEM((CHUNK, DH), jnp.float32),
            pltpu.VMEM((CHUNK, DT), jnp.float32),
            pltpu.VMEM((CHUNK, DH), jnp.float32),
            pltpu.VMEM((CHUNK, DT), jnp.float32),
            pltpu.SemaphoreType.DMA,
            pltpu.SemaphoreType.DMA,
        ],
    )
    return f(gt, ft, gtail, ftail, idx)


def _combine_body(g_ref, gt_ref, f_ref, ft_ref, wg_ref, bg_ref, wf_ref, bf_ref,
                  wa1_ref, wa2_ref, out_ref):
    gh = g_ref[...]
    gt = gt_ref[:, :D_TAIL]
    fh = f_ref[...]
    ft = ft_ref[:, :D_TAIL]
    wg = wg_ref[...]
    wf = wf_ref[...]
    dn = (((1,), (1,)), ((), ()))
    g_out = (lax.dot_general(gh, wg[:, :DH], dn, preferred_element_type=jnp.float32)
             + lax.dot_general(gt, wg[:, DH:], dn, preferred_element_type=jnp.float32)
             + bg_ref[...])
    f_out = (lax.dot_general(fh, wf[:, :DH], dn, preferred_element_type=jnp.float32)
             + lax.dot_general(ft, wf[:, DH:], dn, preferred_element_type=jnp.float32)
             + bf_ref[...])
    wvec = lax.dot_general(wa2_ref[...], wa1_ref[...], (((1,), (0,)), ((), ())),
                           preferred_element_type=jnp.float32)
    diff = jnp.sum((g_out - f_out) * wvec, axis=1, keepdims=True)
    s = 1.0 / (1.0 + jnp.exp(-diff))
    out_ref[...] = jnp.maximum(s * g_out + (1.0 - s) * f_out, 0.0)


def _tc_combine(g_emb, gt_emb, f_emb, ft_emb, Wg, bg, Wf, bf, Wa1, Wa2):
    grid = (N_TOK // BT,)
    return pl.pallas_call(
        _combine_body,
        grid=grid,
        in_specs=[
            pl.BlockSpec((BT, DH), lambda i: (i, 0)),
            pl.BlockSpec((BT, DT), lambda i: (i, 0)),
            pl.BlockSpec((BT, DH), lambda i: (i, 0)),
            pl.BlockSpec((BT, DT), lambda i: (i, 0)),
            pl.BlockSpec((D, D), lambda i: (0, 0)),
            pl.BlockSpec((1, D), lambda i: (0, 0)),
            pl.BlockSpec((D, D), lambda i: (0, 0)),
            pl.BlockSpec((1, D), lambda i: (0, 0)),
            pl.BlockSpec((10, D), lambda i: (0, 0)),
            pl.BlockSpec((1, 10), lambda i: (0, 0)),
        ],
        out_specs=pl.BlockSpec((BT, D), lambda i: (i, 0)),
        out_shape=jax.ShapeDtypeStruct((N_TOK, D), jnp.float32),
    )(g_emb, gt_emb, f_emb, ft_emb, Wg, bg, Wf, bf, Wa1, Wa2)


def kernel(word, glove_table, fasttext_table, Wg, bg, Wf, bf, Wa1, ba1, Wa2, ba2):
    B, L = word.shape
    idx = word.reshape(-1).astype(jnp.int32)
    gtail = jnp.pad(glove_table[:, DH:], ((0, 0), (0, DT - D_TAIL)))
    ftail = jnp.pad(fasttext_table[:, DH:], ((0, 0), (0, DT - D_TAIL)))
    g_emb, gt_emb, f_emb, ft_emb = _sc_gather(glove_table, fasttext_table,
                                              gtail, ftail, idx)
    out = _tc_combine(g_emb, gt_emb, f_emb, ft_emb, Wg, bg.reshape(1, D),
                      Wf, bf.reshape(1, D), Wa1, Wa2)
    return out.reshape(B, L, D)
